# baseline (speedup 1.0000x reference)
import functools

import jax
import jax.numpy as jnp
from jax.experimental import pallas as pl
from jax.experimental.pallas import tpu as pltpu

E = 8
K = 2
H = 1024
F = 1024
T = 2048
BM = 256
TB = T // BM


def _dot_t(a, b):
    return jax.lax.dot_general(a, b, (((1,), (1,)), ((), ())),
                               preferred_element_type=jnp.float32)


def _compute_gate(x, wqkv):
    mix = _dot_t(x, wqkv)
    q = mix[:, 0:E]
    k = mix[:, E:2 * E]
    v = mix[:, 2 * E:3 * E]
    cols = []
    for i in range(E):
        s = q[:, i:i + 1] * k
        m = jnp.max(s, axis=1, keepdims=True)
        p = jnp.exp(s - m)
        p = p / jnp.sum(p, axis=1, keepdims=True)
        cols.append(jnp.sum(p * v, axis=1, keepdims=True))
    logits = jnp.concatenate(cols, axis=1)
    lm = jnp.max(logits, axis=1, keepdims=True)
    pe = jnp.exp(logits - lm)
    probs = pe / jnp.sum(pe, axis=1, keepdims=True)
    idx = jax.lax.broadcasted_iota(jnp.int32, probs.shape, 1)
    m1 = jnp.max(probs, axis=1, keepdims=True)
    i1 = jnp.min(jnp.where(probs == m1, idx, E), axis=1, keepdims=True)
    probs2 = jnp.where(idx == i1, -jnp.inf, probs)
    m2 = jnp.max(probs2, axis=1, keepdims=True)
    i2 = jnp.min(jnp.where(probs2 == m2, idx, E), axis=1, keepdims=True)
    sel = (idx == i1) | (idx == i2)
    return jnp.where(sel, probs, 0.0) / (m1 + m2)


def _moe_kernel(x_ref, wqkv_ref, w1_ref, w3_ref, w2_ref, out_ref, gate_ref):
    e = pl.program_id(0)
    t = pl.program_id(1)
    x = x_ref[pl.ds(t * BM, BM), :]

    @pl.when(e == 0)
    def _router():
        gate_ref[pl.ds(t * BM, BM), :] = _compute_gate(x, wqkv_ref[...])

    xb = x.astype(jnp.bfloat16)
    h1 = _dot_t(xb, w1_ref[0])
    h3 = _dot_t(xb, w3_ref[0])
    h = ((h1 * jax.lax.logistic(h1)) * h3).astype(jnp.bfloat16)
    ye = _dot_t(h, w2_ref[0])

    grows = gate_ref[pl.ds(t * BM, BM), :]
    idx = jax.lax.broadcasted_iota(jnp.int32, grows.shape, 1)
    g = jnp.sum(jnp.where(idx == e, grows, 0.0), axis=1, keepdims=True)

    @pl.when(e == 0)
    def _init():
        out_ref[pl.ds(t * BM, BM), :] = g * ye

    @pl.when(e > 0)
    def _acc():
        out_ref[pl.ds(t * BM, BM), :] += g * ye


@functools.partial(jax.jit, static_argnames=("interpret",))
def kernel(hidden_states, wqkv, w1, w3, w2, interpret=False):
    w1b = w1.astype(jnp.bfloat16)
    w3b = w3.astype(jnp.bfloat16)
    w2b = w2.astype(jnp.bfloat16)
    return pl.pallas_call(
        _moe_kernel,
        grid=(E, TB),
        in_specs=[
            pl.BlockSpec((T, H), lambda e, t: (0, 0)),
            pl.BlockSpec((3 * E, H), lambda e, t: (0, 0)),
            pl.BlockSpec((1, F, H), lambda e, t: (e, 0, 0)),
            pl.BlockSpec((1, F, H), lambda e, t: (e, 0, 0)),
            pl.BlockSpec((1, H, F), lambda e, t: (e, 0, 0)),
        ],
        out_specs=pl.BlockSpec((T, H), lambda e, t: (0, 0)),
        out_shape=jax.ShapeDtypeStruct((T, H), jnp.float32),
        scratch_shapes=[pltpu.VMEM((T, E), jnp.float32)],
        interpret=interpret,
    )(hidden_states, wqkv, w1b, w3b, w2b)


# speedup vs baseline: 1.0188x; 1.0188x over previous
import functools

import jax
import jax.numpy as jnp
from jax import lax
from jax.experimental import pallas as pl
from jax.experimental.pallas import tpu as pltpu
from jax.experimental.pallas import tpu_sc as plsc

E = 8
K = 2
H = 1024
F = 1024
T = 2048
P = K * T
BM = 128
NB = 40
PMAX = NB * BM
BMR = 256
TB = T // BMR

NC = 2
NS = 16
NW = NC * NS
PW = P // NW


def _dot_t(a, b):
    return lax.dot_general(a, b, (((1,), (1,)), ((), ())),
                           preferred_element_type=jnp.float32)


def _router_kernel(x_ref, wqkv_ref, e1_ref, e2_ref, g1_ref, g2_ref):
    x = x_ref[...]
    mix = _dot_t(x, wqkv_ref[...])
    q = mix[:, 0:E]
    k = mix[:, E:2 * E]
    v = mix[:, 2 * E:3 * E]
    cols = []
    for i in range(E):
        s = q[:, i:i + 1] * k
        m = jnp.max(s, axis=1, keepdims=True)
        p = jnp.exp(s - m)
        p = p / jnp.sum(p, axis=1, keepdims=True)
        cols.append(jnp.sum(p * v, axis=1, keepdims=True))
    logits = jnp.concatenate(cols, axis=1)
    lm = jnp.max(logits, axis=1, keepdims=True)
    pe = jnp.exp(logits - lm)
    probs = pe / jnp.sum(pe, axis=1, keepdims=True)
    idx = lax.broadcasted_iota(jnp.int32, probs.shape, 1)
    m1 = jnp.max(probs, axis=1, keepdims=True)
    i1 = jnp.min(jnp.where(probs == m1, idx, E), axis=1, keepdims=True)
    probs2 = jnp.where(idx == i1, -jnp.inf, probs)
    m2 = jnp.max(probs2, axis=1, keepdims=True)
    i2 = jnp.min(jnp.where(probs2 == m2, idx, E), axis=1, keepdims=True)
    den = m1 + m2
    e1_ref[...] = i1
    e2_ref[...] = i2
    g1_ref[...] = m1 / den
    g2_ref[...] = m2 / den


def _router(x, wqkv):
    return pl.pallas_call(
        _router_kernel,
        grid=(TB,),
        in_specs=[
            pl.BlockSpec((BMR, H), lambda t: (t, 0)),
            pl.BlockSpec((3 * E, H), lambda t: (0, 0)),
        ],
        out_specs=[
            pl.BlockSpec((BMR, 1), lambda t: (t, 0)),
            pl.BlockSpec((BMR, 1), lambda t: (t, 0)),
            pl.BlockSpec((BMR, 1), lambda t: (t, 0)),
            pl.BlockSpec((BMR, 1), lambda t: (t, 0)),
        ],
        out_shape=[
            jax.ShapeDtypeStruct((T, 1), jnp.int32),
            jax.ShapeDtypeStruct((T, 1), jnp.int32),
            jax.ShapeDtypeStruct((T, 1), jnp.float32),
            jax.ShapeDtypeStruct((T, 1), jnp.float32),
        ],
    )(x, wqkv)


_SC_MESH = plsc.VectorSubcoreMesh(core_axis_name="c", subcore_axis_name="s")


@functools.partial(
    pl.kernel,
    out_type=[
        jax.ShapeDtypeStruct((PMAX, H), jnp.float32),
        jax.ShapeDtypeStruct((P,), jnp.int32),
        jax.ShapeDtypeStruct((64,), jnp.int32),
    ],
    mesh=_SC_MESH,
    compiler_params=pltpu.CompilerParams(needs_layout_passes=False),
    scratch_types=[
        pltpu.VMEM((P,), jnp.int32),
        pltpu.VMEM((E * 16,), jnp.int32),
        pltpu.VMEM((E * 16,), jnp.int32),
        pltpu.VMEM((PW,), jnp.int32),
        pltpu.VMEM((64,), jnp.int32),
        pltpu.VMEM((16, H), jnp.float32),
        pltpu.SemaphoreType.DMA,
    ],
)
def _dispatch(e_hbm, x_hbm, xp_hbm, pos_hbm, blk_hbm,
              ids_v, ha_v, hp_v, pos_v, blk_v, rows_v, sem):
    wid = lax.axis_index("s") * NC + lax.axis_index("c")
    base = wid * PW
    npref = wid * (PW // 16)
    lane = lax.iota(jnp.int32, 16)
    zero16 = jnp.zeros((16,), jnp.int32)
    ones16 = jnp.ones((16,), jnp.int32)

    pltpu.sync_copy(e_hbm, ids_v)
    for kk in range(E):
        ha_v[pl.ds(kk * 16, 16)] = zero16
        hp_v[pl.ds(kk * 16, 16)] = zero16

    def hist_body(j, carry):
        ids = ids_v[pl.ds(j * 16, 16)]
        addr = ids * 16 + lane
        plsc.addupdate_scatter(ha_v, [addr], ones16)
        m = jnp.full((16,), j < npref)
        plsc.addupdate_scatter(hp_v, [addr], ones16, mask=m)
        return carry

    lax.fori_loop(0, P // 16, hist_body, 0)

    hist = zero16
    pref = zero16
    for e in range(E):
        he = jnp.sum(ha_v[pl.ds(e * 16, 16)])
        pe = jnp.sum(hp_v[pl.ds(e * 16, 16)])
        hist = jnp.where(lane == e, he, hist)
        pref = jnp.where(lane == e, pe, pref)

    padded = ((hist + (BM - 1)) // BM) * BM
    cum = jnp.cumsum(padded)
    gstart = cum - padded
    base_v = gstart + pref

    for j in range(PW // 16):
        ids_j = ids_v[pl.ds(base + j * 16, 16)]
        dest = zero16
        for e in range(E):
            m = ids_j == e
            mi = m.astype(jnp.int32)
            excl = jnp.cumsum(mi) - mi
            be = jnp.sum(jnp.where(lane == e, base_v, 0))
            dest = jnp.where(m, be + excl, dest)
            base_v = jnp.where(lane == e, base_v + jnp.sum(mi), base_v)
        pos_v[pl.ds(j * 16, 16)] = dest
    pltpu.sync_copy(pos_v, pos_hbm.at[pl.ds(base, PW)])

    tot = jnp.sum(padded)
    last_e = jnp.max(jnp.where(padded > 0, lane, 0))
    gs_s = [jnp.sum(jnp.where(lane == e, gstart, 0)) for e in range(E)]
    pd_s = [jnp.sum(jnp.where(lane == e, padded, 0)) for e in range(E)]
    for vblk in range(4):
        r = (lane + vblk * 16) * BM
        acc = zero16
        for e in range(E):
            acc = jnp.where((r >= gs_s[e]) & (r < gs_s[e] + pd_s[e]), e, acc)
        acc = jnp.where(r >= tot, last_e, acc)
        blk_v[pl.ds(vblk * 16, 16)] = acc

    @pl.when(wid == 0)
    def _():
        pltpu.sync_copy(blk_v, blk_hbm)

    tok0 = base % T
    for c in range(PW // 16):
        pltpu.sync_copy(x_hbm.at[pl.ds(tok0 + c * 16, 16)], rows_v)
        idxv = pos_v[pl.ds(c * 16, 16)]
        pltpu.async_copy(rows_v, xp_hbm.at[idxv], sem).wait()


def _gffn_kernel(be_ref, xp_ref, w1_ref, w3_ref, w2_ref, yp_ref):
    del be_ref
    xb = xp_ref[...]
    h1 = _dot_t(xb, w1_ref[0])
    h3 = _dot_t(xb, w3_ref[0])
    h = (h1 * lax.logistic(h1)) * h3
    yp_ref[...] = _dot_t(h, w2_ref[0])


def _gffn(blk, x_perm, w1, w3, w2):
    grid_spec = pltpu.PrefetchScalarGridSpec(
        num_scalar_prefetch=1,
        grid=(NB,),
        in_specs=[
            pl.BlockSpec((BM, H), lambda b, s: (b, 0)),
            pl.BlockSpec((1, F, H), lambda b, s: (s[b], 0, 0)),
            pl.BlockSpec((1, F, H), lambda b, s: (s[b], 0, 0)),
            pl.BlockSpec((1, H, F), lambda b, s: (s[b], 0, 0)),
        ],
        out_specs=pl.BlockSpec((BM, H), lambda b, s: (b, 0)),
    )
    return pl.pallas_call(
        _gffn_kernel,
        grid_spec=grid_spec,
        out_shape=jax.ShapeDtypeStruct((PMAX, H), jnp.float32),
    )(blk, x_perm, w1, w3, w2)


@functools.partial(
    pl.kernel,
    out_type=jax.ShapeDtypeStruct((P, H), jnp.float32),
    mesh=_SC_MESH,
    compiler_params=pltpu.CompilerParams(needs_layout_passes=False),
    scratch_types=[
        pltpu.VMEM((PW,), jnp.int32),
        pltpu.VMEM((16, H), jnp.float32),
        pltpu.SemaphoreType.DMA,
    ],
)
def _undispatch(pos_hbm, yp_hbm, z_hbm, pos_v, rows_v, sem):
    wid = lax.axis_index("s") * NC + lax.axis_index("c")
    base = wid * PW
    pltpu.sync_copy(pos_hbm.at[pl.ds(base, PW)], pos_v)
    for c in range(PW // 16):
        idxv = pos_v[pl.ds(c * 16, 16)]
        pltpu.async_copy(yp_hbm.at[idxv], rows_v, sem).wait()
        pltpu.sync_copy(rows_v, z_hbm.at[pl.ds(base + c * 16, 16)])


def _combine_kernel(z_ref, g1_ref, g2_ref, out_ref):
    out_ref[...] = g1_ref[...] * z_ref[0] + g2_ref[...] * z_ref[1]


def _combine(z2, g1, g2):
    return pl.pallas_call(
        _combine_kernel,
        grid=(TB,),
        in_specs=[
            pl.BlockSpec((2, BMR, H), lambda t: (0, t, 0)),
            pl.BlockSpec((BMR, 1), lambda t: (t, 0)),
            pl.BlockSpec((BMR, 1), lambda t: (t, 0)),
        ],
        out_specs=pl.BlockSpec((BMR, H), lambda t: (t, 0)),
        out_shape=jax.ShapeDtypeStruct((T, H), jnp.float32),
    )(z2, g1, g2)


@jax.jit
def kernel(hidden_states, wqkv, w1, w3, w2):
    e1, e2, g1, g2 = _router(hidden_states, wqkv)
    e_all = jnp.concatenate([e1[:, 0], e2[:, 0]])
    x_perm, pos, blk = _dispatch(e_all, hidden_states)
    y_perm = _gffn(blk, x_perm, w1, w3, w2)
    z = _undispatch(pos, y_perm)
    return _combine(z.reshape(2, T, H), g1, g2)


# speedup vs baseline: 1.1485x; 1.1274x over previous
import functools

import jax
import jax.numpy as jnp
from jax import lax
from jax.experimental import pallas as pl
from jax.experimental.pallas import tpu as pltpu
from jax.experimental.pallas import tpu_sc as plsc

E = 8
K = 2
H = 1024
F = 1024
T = 2048
P = K * T
BM = 128
NB = 40
PMAX = NB * BM
BMR = 256
TB = T // BMR

NC = 2
NS = 16
NW = NC * NS
PW = P // NW
TW = T // NW


def _dot_t(a, b):
    return lax.dot_general(a, b, (((1,), (1,)), ((), ())),
                           preferred_element_type=jnp.float32)


def _router_kernel(x_ref, wqkv_ref, e1_ref, e2_ref, g1_ref, g2_ref):
    x = x_ref[...]
    mix = _dot_t(x, wqkv_ref[...])
    q = mix[:, 0:E]
    k = mix[:, E:2 * E]
    v = mix[:, 2 * E:3 * E]
    cols = []
    for i in range(E):
        s = q[:, i:i + 1] * k
        m = jnp.max(s, axis=1, keepdims=True)
        p = jnp.exp(s - m)
        p = p / jnp.sum(p, axis=1, keepdims=True)
        cols.append(jnp.sum(p * v, axis=1, keepdims=True))
    logits = jnp.concatenate(cols, axis=1)
    lm = jnp.max(logits, axis=1, keepdims=True)
    pe = jnp.exp(logits - lm)
    probs = pe / jnp.sum(pe, axis=1, keepdims=True)
    idx = lax.broadcasted_iota(jnp.int32, probs.shape, 1)
    m1 = jnp.max(probs, axis=1, keepdims=True)
    i1 = jnp.min(jnp.where(probs == m1, idx, E), axis=1, keepdims=True)
    probs2 = jnp.where(idx == i1, -jnp.inf, probs)
    m2 = jnp.max(probs2, axis=1, keepdims=True)
    i2 = jnp.min(jnp.where(probs2 == m2, idx, E), axis=1, keepdims=True)
    den = m1 + m2
    e1_ref[...] = i1
    e2_ref[...] = i2
    g1_ref[...] = m1 / den
    g2_ref[...] = m2 / den


def _router(x, wqkv):
    return pl.pallas_call(
        _router_kernel,
        grid=(TB,),
        in_specs=[
            pl.BlockSpec((BMR, H), lambda t: (t, 0)),
            pl.BlockSpec((3 * E, H), lambda t: (0, 0)),
        ],
        out_specs=[
            pl.BlockSpec((BMR, 1), lambda t: (t, 0)),
            pl.BlockSpec((BMR, 1), lambda t: (t, 0)),
            pl.BlockSpec((BMR, 1), lambda t: (t, 0)),
            pl.BlockSpec((BMR, 1), lambda t: (t, 0)),
        ],
        out_shape=[
            jax.ShapeDtypeStruct((T, 1), jnp.int32),
            jax.ShapeDtypeStruct((T, 1), jnp.int32),
            jax.ShapeDtypeStruct((T, 1), jnp.float32),
            jax.ShapeDtypeStruct((T, 1), jnp.float32),
        ],
    )(x, wqkv)


_SC_MESH = plsc.VectorSubcoreMesh(core_axis_name="c", subcore_axis_name="s")


@functools.partial(
    pl.kernel,
    out_type=[
        jax.ShapeDtypeStruct((PMAX, H), jnp.float32),
        jax.ShapeDtypeStruct((P,), jnp.int32),
        jax.ShapeDtypeStruct((64,), jnp.int32),
    ],
    mesh=_SC_MESH,
    compiler_params=pltpu.CompilerParams(needs_layout_passes=False),
    scratch_types=[
        pltpu.VMEM((P,), jnp.int32),
        pltpu.VMEM((E * 16,), jnp.int32),
        pltpu.VMEM((E * 16,), jnp.int32),
        pltpu.VMEM((PW,), jnp.int32),
        pltpu.VMEM((64,), jnp.int32),
        pltpu.VMEM((2, 16, H), jnp.float32),
        pltpu.SemaphoreType.DMA,
        pltpu.SemaphoreType.DMA,
    ],
)
def _dispatch(e_hbm, x_hbm, xp_hbm, pos_hbm, blk_hbm,
              ids_v, ha_v, hp_v, pos_v, blk_v, rows_v,
              sem_in, sem_out):
    wid = lax.axis_index("s") * NC + lax.axis_index("c")
    base = wid * PW
    npref = wid * (PW // 16)
    lane = lax.iota(jnp.int32, 16)
    zero16 = jnp.zeros((16,), jnp.int32)
    ones16 = jnp.ones((16,), jnp.int32)
    tok0 = base % T
    nch = PW // 16

    h_in = [None] * nch
    h_in[0] = pltpu.async_copy(x_hbm.at[pl.ds(tok0, 16)], rows_v.at[0], sem_in)
    h_in[1] = pltpu.async_copy(x_hbm.at[pl.ds(tok0 + 16, 16)], rows_v.at[1],
                               sem_in)
    pltpu.sync_copy(e_hbm, ids_v)
    for kk in range(E):
        ha_v[pl.ds(kk * 16, 16)] = zero16
        hp_v[pl.ds(kk * 16, 16)] = zero16

    def hist_body(j, carry):
        ids = ids_v[pl.ds(j * 16, 16)]
        addr = ids * 16 + lane
        plsc.addupdate_scatter(ha_v, [addr], ones16)
        m = jnp.full((16,), j < npref)
        plsc.addupdate_scatter(hp_v, [addr], ones16, mask=m)
        return carry

    lax.fori_loop(0, P // 16, hist_body, 0)

    hist = zero16
    pref = zero16
    for e in range(E):
        he = jnp.sum(ha_v[pl.ds(e * 16, 16)])
        pe = jnp.sum(hp_v[pl.ds(e * 16, 16)])
        hist = jnp.where(lane == e, he, hist)
        pref = jnp.where(lane == e, pe, pref)

    padded = ((hist + (BM - 1)) // BM) * BM
    cum = jnp.cumsum(padded)
    gstart = cum - padded
    base_v = gstart + pref

    for j in range(nch):
        ids_j = ids_v[pl.ds(base + j * 16, 16)]
        dest = zero16
        for e in range(E):
            m = ids_j == e
            mi = m.astype(jnp.int32)
            excl = jnp.cumsum(mi) - mi
            be = jnp.sum(jnp.where(lane == e, base_v, 0))
            dest = jnp.where(m, be + excl, dest)
            base_v = jnp.where(lane == e, base_v + jnp.sum(mi), base_v)
        pos_v[pl.ds(j * 16, 16)] = dest
    pltpu.sync_copy(pos_v, pos_hbm.at[pl.ds(base, PW)])

    tot = jnp.sum(padded)
    nblk = tot // BM
    last_e = jnp.max(jnp.where(padded > 0, lane, 0))
    gs_s = [jnp.sum(jnp.where(lane == e, gstart, 0)) for e in range(E)]
    pd_s = [jnp.sum(jnp.where(lane == e, padded, 0)) for e in range(E)]
    for vblk in range(4):
        r = (lane + vblk * 16) * BM
        acc = zero16
        for e in range(E):
            acc = jnp.where((r >= gs_s[e]) & (r < gs_s[e] + pd_s[e]), e, acc)
        acc = jnp.where(r >= tot, last_e, acc)
        if vblk == 3:
            acc = jnp.where(lane == 0, nblk, acc)
        blk_v[pl.ds(vblk * 16, 16)] = acc

    @pl.when(wid == 0)
    def _():
        pltpu.sync_copy(blk_v, blk_hbm)

    h_out = [None] * nch
    for c in range(nch):
        b = c % 2
        h_in[c].wait()
        idxv = pos_v[pl.ds(c * 16, 16)]
        h_out[c] = pltpu.async_copy(rows_v.at[b], xp_hbm.at[idxv], sem_out)
        if c + 2 < nch:
            h_out[c].wait()
            h_in[c + 2] = pltpu.async_copy(
                x_hbm.at[pl.ds(tok0 + (c + 2) * 16, 16)], rows_v.at[b],
                sem_in)
    h_out[nch - 2].wait()
    h_out[nch - 1].wait()


def _gffn_kernel(be_ref, xp_ref, w1_ref, w3_ref, w2_ref, yp_ref):
    b = pl.program_id(0)

    @pl.when(b < be_ref[48])
    def _():
        xb = xp_ref[...]
        h1 = _dot_t(xb, w1_ref[0])
        h3 = _dot_t(xb, w3_ref[0])
        h = (h1 * lax.logistic(h1)) * h3
        yp_ref[...] = _dot_t(h, w2_ref[0])


def _gffn(blk, x_perm, w1, w3, w2):
    def _clamp(b, s):
        return jnp.minimum(b, s[48] - 1)

    grid_spec = pltpu.PrefetchScalarGridSpec(
        num_scalar_prefetch=1,
        grid=(NB,),
        in_specs=[
            pl.BlockSpec((BM, H), lambda b, s: (_clamp(b, s), 0)),
            pl.BlockSpec((1, F, H), lambda b, s: (s[b], 0, 0)),
            pl.BlockSpec((1, F, H), lambda b, s: (s[b], 0, 0)),
            pl.BlockSpec((1, H, F), lambda b, s: (s[b], 0, 0)),
        ],
        out_specs=pl.BlockSpec((BM, H), lambda b, s: (_clamp(b, s), 0)),
    )
    return pl.pallas_call(
        _gffn_kernel,
        grid_spec=grid_spec,
        out_shape=jax.ShapeDtypeStruct((PMAX, H), jnp.float32),
    )(blk, x_perm, w1, w3, w2)


@functools.partial(
    pl.kernel,
    out_type=jax.ShapeDtypeStruct((T, H), jnp.float32),
    mesh=_SC_MESH,
    compiler_params=pltpu.CompilerParams(needs_layout_passes=False),
    scratch_types=[
        pltpu.VMEM((TW,), jnp.int32),
        pltpu.VMEM((TW,), jnp.int32),
        pltpu.VMEM((TW,), jnp.float32),
        pltpu.VMEM((TW,), jnp.float32),
        pltpu.VMEM((2, 16, H), jnp.float32),
        pltpu.VMEM((2, 16, H), jnp.float32),
        pltpu.VMEM((2, 16, H), jnp.float32),
        pltpu.SemaphoreType.DMA,
        pltpu.SemaphoreType.DMA,
    ],
)
def _undispatch(pos_hbm, g1_hbm, g2_hbm, yp_hbm, out_hbm,
                pos0_v, pos1_v, g1_v, g2_v, r0_v, r1_v, o_v,
                sem_in, sem_out):
    wid = lax.axis_index("s") * NC + lax.axis_index("c")
    t0 = wid * TW
    nch = TW // 16
    lane = lax.iota(jnp.int32, 16)
    pltpu.sync_copy(pos_hbm.at[pl.ds(t0, TW)], pos0_v)
    pltpu.sync_copy(pos_hbm.at[pl.ds(T + t0, TW)], pos1_v)
    pltpu.sync_copy(g1_hbm.at[pl.ds(t0, TW)], g1_v)
    pltpu.sync_copy(g2_hbm.at[pl.ds(t0, TW)], g2_v)

    def _fire(c, b):
        i0 = pos0_v[pl.ds(c * 16, 16)]
        i1 = pos1_v[pl.ds(c * 16, 16)]
        return (pltpu.async_copy(yp_hbm.at[i0], r0_v.at[b], sem_in),
                pltpu.async_copy(yp_hbm.at[i1], r1_v.at[b], sem_in))

    h_in = [None] * nch
    h_out = [None] * nch
    h_in[0] = _fire(0, 0)
    if nch > 1:
        h_in[1] = _fire(1, 1)
    for c in range(nch):
        b = c % 2
        h_in[c][0].wait()
        h_in[c][1].wait()
        if c >= 2:
            h_out[c - 2].wait()
        gv1 = g1_v[pl.ds(c * 16, 16)]
        gv2 = g2_v[pl.ds(c * 16, 16)]

        def row_body(i, carry, _b=b, _g1=gv1, _g2=gv2):
            a1 = jnp.sum(jnp.where(lane == i, _g1, 0.0))
            a2 = jnp.sum(jnp.where(lane == i, _g2, 0.0))
            for hc in range(H // 16):
                sl = pl.ds(hc * 16, 16)
                o_v[_b, i, sl] = a1 * r0_v[_b, i, sl] + a2 * r1_v[_b, i, sl]
            return carry

        lax.fori_loop(0, 16, row_body, 0)
        if c + 2 < nch:
            h_in[c + 2] = _fire(c + 2, b)
        h_out[c] = pltpu.async_copy(
            o_v.at[b], out_hbm.at[pl.ds(t0 + c * 16, 16)], sem_out)
    h_out[nch - 1].wait()
    if nch > 1:
        h_out[nch - 2].wait()


@jax.jit
def kernel(hidden_states, wqkv, w1, w3, w2):
    e1, e2, g1, g2 = _router(hidden_states, wqkv)
    e_all = jnp.concatenate([e1[:, 0], e2[:, 0]])
    x_perm, pos, blk = _dispatch(e_all, hidden_states)
    y_perm = _gffn(blk, x_perm, w1, w3, w2)
    return _undispatch(pos, g1[:, 0], g2[:, 0], y_perm)
